# Initial kernel scaffold; baseline (speedup 1.0000x reference)
import jax
import jax.numpy as jnp
from jax.experimental import pallas as pl


def kernel(x, embed_w, gate_w, gate_b, w1, b1, w2, b2, ln_g, ln_b, head_w, head_b):
    raise NotImplementedError("write your pallas kernel here")


# speedup vs baseline: 3.2765x; 3.2765x over previous
import jax
import jax.numpy as jnp
from jax import lax
from jax.experimental import pallas as pl
from jax.experimental.pallas import tpu as pltpu
from jax.experimental.pallas import tpu_sc as plsc

_D = 768
_E = 64
_K = 2
_B = 4
_S = 8192
_NC = 2
_NS = 16
_NW = _NC * _NS
_TPW = (_B * _S) // _NW
_CH = 64
_NCH = _TPW // _CH
_LANES = 16
_DC = _D // _LANES
_HCH = 512
_NHC = (2 * _D) // _HCH


def _emb_body(x_hbm, tab_hbm, out_hbm, idx_v, rows_v, acc_v, sem0, sem1):
    wid = lax.axis_index("s") * _NC + lax.axis_index("c")
    pltpu.sync_copy(x_hbm.at[wid], idx_v)

    def _zero(d, carry):
        acc_v[pl.ds(d * _LANES, _LANES)] = jnp.zeros((_LANES,), jnp.float32)
        return carry

    lax.fori_loop(0, _DC, _zero, 0)

    sems = (sem0, sem1)

    def _start(ch):
        return pltpu.async_copy(
            tab_hbm.at[idx_v.at[ch]], rows_v.at[ch % 2], sems[ch % 2])

    pending = {0: _start(0)}
    for ch in range(_NCH):
        if ch + 1 < _NCH:
            pending[ch + 1] = _start(ch + 1)
        pending.pop(ch).wait()
        buf = ch % 2

        def _accum(d, carry, buf=buf):
            base = d * _LANES
            a = acc_v[pl.ds(base, _LANES)]
            for r in range(_CH):
                a = a + rows_v[buf, r, pl.ds(base, _LANES)]
            acc_v[pl.ds(base, _LANES)] = a
            return carry

        lax.fori_loop(0, _DC, _accum, 0)

    pltpu.sync_copy(acc_v, out_hbm.at[wid])


def _router_body(ps_ref, gw_ref, gb_ref, h_ref, vals_ref, idx_ref):
    h = jnp.sum(ps_ref[...], axis=1) * (1.0 / _S)
    logits = lax.dot_general(h, gw_ref[...], (((1,), (0,)), ((), ())),
                             preferred_element_type=jnp.float32)
    logits = logits + gb_ref[...]
    m = jnp.max(logits, axis=-1, keepdims=True)
    ex = jnp.exp(logits - m)
    sm = ex / jnp.sum(ex, axis=-1, keepdims=True)
    iota = lax.broadcasted_iota(jnp.int32, (_B, _E), 1)
    v1 = jnp.max(sm, axis=-1, keepdims=True)
    i1 = jnp.min(jnp.where(sm == v1, iota, _E), axis=-1, keepdims=True)
    sm2 = jnp.where(iota == i1, -jnp.inf, sm)
    v2 = jnp.max(sm2, axis=-1, keepdims=True)
    i2 = jnp.min(jnp.where(sm2 == v2, iota, _E), axis=-1, keepdims=True)
    h_ref[...] = h
    vals_ref[...] = jnp.concatenate([v1, v2], axis=1)
    idx_ref[...] = jnp.concatenate([i1, i2], axis=1)


def _moe_body(ti_ref, h_ref, vals_ref, w1_ref, b1_ref, w2_ref, b2_ref,
              lng_ref, lnb_ref, hw_ref, hb_ref, out_ref, comb):
    t = pl.program_id(0)
    c = pl.program_id(1)

    @pl.when((t == 0) & (c == 0))
    def _():
        comb[...] = jnp.zeros((_B, _D), jnp.float32)

    b = t // _K
    k = t % _K
    riota = lax.broadcasted_iota(jnp.int32, (_B, 1), 0)
    hsel = jnp.sum(jnp.where(riota == b, h_ref[...], 0.0),
                   axis=0, keepdims=True)
    hid = lax.dot_general(hsel, w1_ref[0], (((1,), (0,)), ((), ())),
                          preferred_element_type=jnp.float32) + b1_ref[0, 0]
    hid = jnp.maximum(hid, 0.0)
    contrib = lax.dot_general(hid, w2_ref[0], (((1,), (0,)), ((), ())),
                              preferred_element_type=jnp.float32)
    contrib = contrib + jnp.where(c == 0, 1.0, 0.0) * b2_ref[0]
    ciota = lax.broadcasted_iota(jnp.int32, (_B, _K), 1)
    v = jnp.sum(jnp.where((riota == b) & (ciota == k), vals_ref[...], 0.0))
    comb[...] = comb[...] + jnp.where(riota == b, 1.0, 0.0) * (v * contrib)

    @pl.when((t == _B * _K - 1) & (c == _NHC - 1))
    def _():
        xx = comb[...]
        mu = jnp.mean(xx, axis=-1, keepdims=True)
        var = jnp.mean((xx - mu) ** 2, axis=-1, keepdims=True)
        normed = (xx - mu) * lax.rsqrt(var + 1e-5) * lng_ref[...] + lnb_ref[...]
        out_ref[...] = lax.dot_general(
            normed, hw_ref[...], (((1,), (0,)), ((), ())),
            preferred_element_type=jnp.float32) + hb_ref[...]


def kernel(x, embed_w, gate_w, gate_b, w1, b1, w2, b2, ln_g, ln_b, head_w, head_b):
    x_resh = x.astype(jnp.int32).reshape(_NW, _NCH, _CH)

    emb_call = pl.kernel(
        _emb_body,
        out_type=jax.ShapeDtypeStruct((_NW, _D), jnp.float32),
        mesh=plsc.VectorSubcoreMesh(core_axis_name="c", subcore_axis_name="s"),
        scratch_types=[
            pltpu.VMEM((_NCH, _CH), jnp.int32),
            pltpu.VMEM((2, _CH, _D), jnp.float32),
            pltpu.VMEM((_D,), jnp.float32),
            pltpu.SemaphoreType.DMA,
            pltpu.SemaphoreType.DMA,
        ],
    )
    psums = emb_call(x_resh, embed_w)

    h, vals, ti = pl.pallas_call(
        _router_body,
        out_shape=[
            jax.ShapeDtypeStruct((_B, _D), jnp.float32),
            jax.ShapeDtypeStruct((_B, _K), jnp.float32),
            jax.ShapeDtypeStruct((_B, _K), jnp.int32),
        ],
    )(psums.reshape(_B, _NW // _B, _D), gate_w, gate_b.reshape(1, _E))

    grid_spec = pltpu.PrefetchScalarGridSpec(
        num_scalar_prefetch=1,
        grid=(_B * _K, _NHC),
        in_specs=[
            pl.BlockSpec((_B, _D), lambda t, c, ti: (0, 0)),
            pl.BlockSpec((_B, _K), lambda t, c, ti: (0, 0)),
            pl.BlockSpec((1, _D, _HCH), lambda t, c, ti: (ti[t], 0, c)),
            pl.BlockSpec((1, 1, 1, _HCH), lambda t, c, ti: (ti[t], c, 0, 0)),
            pl.BlockSpec((1, _HCH, _D), lambda t, c, ti: (ti[t], c, 0)),
            pl.BlockSpec((1, 1, _D), lambda t, c, ti: (ti[t], 0, 0)),
            pl.BlockSpec((1, _D), lambda t, c, ti: (0, 0)),
            pl.BlockSpec((1, _D), lambda t, c, ti: (0, 0)),
            pl.BlockSpec((_D, 20), lambda t, c, ti: (0, 0)),
            pl.BlockSpec((1, 20), lambda t, c, ti: (0, 0)),
        ],
        out_specs=pl.BlockSpec((_B, 20), lambda t, c, ti: (0, 0)),
        scratch_shapes=[pltpu.VMEM((_B, _D), jnp.float32)],
    )
    out_logits = pl.pallas_call(
        _moe_body,
        grid_spec=grid_spec,
        out_shape=jax.ShapeDtypeStruct((_B, 20), jnp.float32),
    )(ti.reshape(_B * _K), h, vals, w1, b1.reshape(_E, _NHC, 1, _HCH), w2,
      b2.reshape(_E, 1, _D),
      ln_g.reshape(1, _D), ln_b.reshape(1, _D), head_w, head_b.reshape(1, 20))

    return (out_logits, ti)


# speedup vs baseline: 4.0224x; 1.2276x over previous
import jax
import jax.numpy as jnp
from jax import lax
from jax.experimental import pallas as pl
from jax.experimental.pallas import tpu as pltpu
from jax.experimental.pallas import tpu_sc as plsc

_D = 768
_E = 64
_K = 2
_B = 4
_S = 8192
_NC = 2
_NS = 16
_NW = _NC * _NS
_TPW = (_B * _S) // _NW
_CH = 64
_NCH = _TPW // _CH
_LANES = 16
_DC = _D // _LANES
_HCH = 512
_NHC = (2 * _D) // _HCH


def _emb_body(x_hbm, tab_hbm, out_hbm, idx_v, rows_v, acc_v, sem0, sem1):
    wid = lax.axis_index("s") * _NC + lax.axis_index("c")
    pltpu.sync_copy(x_hbm.at[wid], idx_v)

    def _zero(d, carry):
        acc_v[pl.ds(d * _LANES, _LANES)] = jnp.zeros((_LANES,), jnp.float32)
        return carry

    lax.fori_loop(0, _DC, _zero, 0)

    sems = (sem0, sem1)

    def _start(ch):
        return pltpu.async_copy(
            tab_hbm.at[idx_v.at[ch]], rows_v.at[ch % 2], sems[ch % 2])

    pending = {0: _start(0)}
    for ch in range(_NCH):
        if ch + 1 < _NCH:
            pending[ch + 1] = _start(ch + 1)
        pending.pop(ch).wait()
        buf = ch % 2

        def _accum(d, carry, buf=buf):
            base = d * _LANES
            lanes = [rows_v[buf, r, pl.ds(base, _LANES)] for r in range(_CH)]
            parts = []
            for g in range(4):
                a = lanes[g * (_CH // 4)]
                for r in range(g * (_CH // 4) + 1, (g + 1) * (_CH // 4)):
                    a = a + lanes[r]
                parts.append(a)
            acc_v[pl.ds(base, _LANES)] = (
                acc_v[pl.ds(base, _LANES)] + ((parts[0] + parts[1]) + (parts[2] + parts[3])))
            return carry

        lax.fori_loop(0, _DC, _accum, 0)

    pltpu.sync_copy(acc_v, out_hbm.at[wid])


def _router_body(ps_ref, gw_ref, gb_ref, h_ref, vals_ref, idx_ref):
    h = jnp.sum(ps_ref[...], axis=1) * (1.0 / _S)
    logits = lax.dot_general(h, gw_ref[...], (((1,), (0,)), ((), ())),
                             preferred_element_type=jnp.float32)
    logits = logits + gb_ref[...]
    m = jnp.max(logits, axis=-1, keepdims=True)
    ex = jnp.exp(logits - m)
    sm = ex / jnp.sum(ex, axis=-1, keepdims=True)
    iota = lax.broadcasted_iota(jnp.int32, (_B, _E), 1)
    v1 = jnp.max(sm, axis=-1, keepdims=True)
    i1 = jnp.min(jnp.where(sm == v1, iota, _E), axis=-1, keepdims=True)
    sm2 = jnp.where(iota == i1, -jnp.inf, sm)
    v2 = jnp.max(sm2, axis=-1, keepdims=True)
    i2 = jnp.min(jnp.where(sm2 == v2, iota, _E), axis=-1, keepdims=True)
    h_ref[...] = h
    vals_ref[...] = jnp.concatenate([v1, v2], axis=1)
    idx_ref[...] = jnp.concatenate([i1, i2], axis=1)


def _moe_body(ti_ref, h_ref, vals_ref, w1_ref, b1_ref, w2_ref, b2_ref,
              lng_ref, lnb_ref, hw_ref, hb_ref, out_ref, comb):
    t = pl.program_id(0)
    c = pl.program_id(1)

    @pl.when((t == 0) & (c == 0))
    def _():
        comb[...] = jnp.zeros((_B, _D), jnp.float32)

    b = t // _K
    k = t % _K
    riota = lax.broadcasted_iota(jnp.int32, (_B, 1), 0)
    hsel = jnp.sum(jnp.where(riota == b, h_ref[...], 0.0),
                   axis=0, keepdims=True)
    hid = lax.dot_general(hsel, w1_ref[0], (((1,), (0,)), ((), ())),
                          preferred_element_type=jnp.float32) + b1_ref[0, 0]
    hid = jnp.maximum(hid, 0.0)
    contrib = lax.dot_general(hid, w2_ref[0], (((1,), (0,)), ((), ())),
                              preferred_element_type=jnp.float32)
    contrib = contrib + jnp.where(c == 0, 1.0, 0.0) * b2_ref[0]
    ciota = lax.broadcasted_iota(jnp.int32, (_B, _K), 1)
    v = jnp.sum(jnp.where((riota == b) & (ciota == k), vals_ref[...], 0.0))
    comb[...] = comb[...] + jnp.where(riota == b, 1.0, 0.0) * (v * contrib)

    @pl.when((t == _B * _K - 1) & (c == _NHC - 1))
    def _():
        xx = comb[...]
        mu = jnp.mean(xx, axis=-1, keepdims=True)
        var = jnp.mean((xx - mu) ** 2, axis=-1, keepdims=True)
        normed = (xx - mu) * lax.rsqrt(var + 1e-5) * lng_ref[...] + lnb_ref[...]
        out_ref[...] = lax.dot_general(
            normed, hw_ref[...], (((1,), (0,)), ((), ())),
            preferred_element_type=jnp.float32) + hb_ref[...]


def kernel(x, embed_w, gate_w, gate_b, w1, b1, w2, b2, ln_g, ln_b, head_w, head_b):
    x_resh = x.astype(jnp.int32).reshape(_NW, _NCH, _CH)

    emb_call = pl.kernel(
        _emb_body,
        out_type=jax.ShapeDtypeStruct((_NW, _D), jnp.float32),
        mesh=plsc.VectorSubcoreMesh(core_axis_name="c", subcore_axis_name="s"),
        scratch_types=[
            pltpu.VMEM((_NCH, _CH), jnp.int32),
            pltpu.VMEM((2, _CH, _D), jnp.float32),
            pltpu.VMEM((_D,), jnp.float32),
            pltpu.SemaphoreType.DMA,
            pltpu.SemaphoreType.DMA,
        ],
    )
    psums = emb_call(x_resh, embed_w)

    h, vals, ti = pl.pallas_call(
        _router_body,
        out_shape=[
            jax.ShapeDtypeStruct((_B, _D), jnp.float32),
            jax.ShapeDtypeStruct((_B, _K), jnp.float32),
            jax.ShapeDtypeStruct((_B, _K), jnp.int32),
        ],
    )(psums.reshape(_B, _NW // _B, _D), gate_w, gate_b.reshape(1, _E))

    grid_spec = pltpu.PrefetchScalarGridSpec(
        num_scalar_prefetch=1,
        grid=(_B * _K, _NHC),
        in_specs=[
            pl.BlockSpec((_B, _D), lambda t, c, ti: (0, 0)),
            pl.BlockSpec((_B, _K), lambda t, c, ti: (0, 0)),
            pl.BlockSpec((1, _D, _HCH), lambda t, c, ti: (ti[t], 0, c)),
            pl.BlockSpec((1, 1, 1, _HCH), lambda t, c, ti: (ti[t], c, 0, 0)),
            pl.BlockSpec((1, _HCH, _D), lambda t, c, ti: (ti[t], c, 0)),
            pl.BlockSpec((1, 1, _D), lambda t, c, ti: (ti[t], 0, 0)),
            pl.BlockSpec((1, _D), lambda t, c, ti: (0, 0)),
            pl.BlockSpec((1, _D), lambda t, c, ti: (0, 0)),
            pl.BlockSpec((_D, 20), lambda t, c, ti: (0, 0)),
            pl.BlockSpec((1, 20), lambda t, c, ti: (0, 0)),
        ],
        out_specs=pl.BlockSpec((_B, 20), lambda t, c, ti: (0, 0)),
        scratch_shapes=[pltpu.VMEM((_B, _D), jnp.float32)],
    )
    out_logits = pl.pallas_call(
        _moe_body,
        grid_spec=grid_spec,
        out_shape=jax.ShapeDtypeStruct((_B, 20), jnp.float32),
    )(ti.reshape(_B * _K), h, vals, w1, b1.reshape(_E, _NHC, 1, _HCH), w2,
      b2.reshape(_E, 1, _D),
      ln_g.reshape(1, _D), ln_b.reshape(1, _D), head_w, head_b.reshape(1, 20))

    return (out_logits, ti)


# speedup vs baseline: 4.2531x; 1.0574x over previous
import jax
import jax.numpy as jnp
from jax import lax
from jax.experimental import pallas as pl
from jax.experimental.pallas import tpu as pltpu
from jax.experimental.pallas import tpu_sc as plsc

_D = 768
_E = 64
_K = 2
_B = 4
_S = 8192
_NC = 2
_NS = 16
_NW = _NC * _NS
_TPW = (_B * _S) // _NW
_CH = 64
_NCH = _TPW // _CH
_LANES = 16
_DC = _D // _LANES
_HCH = 1536
_NHC = (2 * _D) // _HCH


def _emb_body(x_hbm, tab_hbm, out_hbm, idx_v, rows_v, acc_v, sem0, sem1):
    wid = lax.axis_index("s") * _NC + lax.axis_index("c")
    pltpu.sync_copy(x_hbm.at[wid], idx_v)

    def _zero(d, carry):
        acc_v[pl.ds(d * _LANES, _LANES)] = jnp.zeros((_LANES,), jnp.float32)
        return carry

    lax.fori_loop(0, _DC, _zero, 0)

    sems = (sem0, sem1)

    def _start(ch):
        return pltpu.async_copy(
            tab_hbm.at[idx_v.at[ch]], rows_v.at[ch % 2], sems[ch % 2])

    pending = {0: _start(0)}
    for ch in range(_NCH):
        if ch + 1 < _NCH:
            pending[ch + 1] = _start(ch + 1)
        pending.pop(ch).wait()
        buf = ch % 2

        def _accum(d, carry, buf=buf):
            base = d * _LANES
            lanes = [rows_v[buf, r, pl.ds(base, _LANES)] for r in range(_CH)]
            parts = []
            for g in range(4):
                a = lanes[g * (_CH // 4)]
                for r in range(g * (_CH // 4) + 1, (g + 1) * (_CH // 4)):
                    a = a + lanes[r]
                parts.append(a)
            acc_v[pl.ds(base, _LANES)] = (
                acc_v[pl.ds(base, _LANES)] + ((parts[0] + parts[1]) + (parts[2] + parts[3])))
            return carry

        lax.fori_loop(0, _DC, _accum, 0)

    pltpu.sync_copy(acc_v, out_hbm.at[wid])


def _router_body(ps_ref, gw_ref, gb_ref, h_ref, vals_ref, idx_ref):
    h = jnp.sum(ps_ref[...], axis=1) * (1.0 / _S)
    logits = lax.dot_general(h, gw_ref[...], (((1,), (0,)), ((), ())),
                             preferred_element_type=jnp.float32)
    logits = logits + gb_ref[...]
    m = jnp.max(logits, axis=-1, keepdims=True)
    ex = jnp.exp(logits - m)
    sm = ex / jnp.sum(ex, axis=-1, keepdims=True)
    iota = lax.broadcasted_iota(jnp.int32, (_B, _E), 1)
    v1 = jnp.max(sm, axis=-1, keepdims=True)
    i1 = jnp.min(jnp.where(sm == v1, iota, _E), axis=-1, keepdims=True)
    sm2 = jnp.where(iota == i1, -jnp.inf, sm)
    v2 = jnp.max(sm2, axis=-1, keepdims=True)
    i2 = jnp.min(jnp.where(sm2 == v2, iota, _E), axis=-1, keepdims=True)
    h_ref[...] = h
    vals_ref[...] = jnp.concatenate([v1, v2], axis=1)
    idx_ref[...] = jnp.concatenate([i1, i2], axis=1)


def _moe_body(ti_ref, h_ref, vals_ref, w1_ref, b1_ref, w2_ref, b2_ref,
              lng_ref, lnb_ref, hw_ref, hb_ref, out_ref, comb):
    t = pl.program_id(0)
    c = pl.program_id(1)

    @pl.when((t == 0) & (c == 0))
    def _():
        comb[...] = jnp.zeros((_B, _D), jnp.float32)

    b = t // _K
    k = t % _K
    riota = lax.broadcasted_iota(jnp.int32, (_B, 1), 0)
    hsel = jnp.sum(jnp.where(riota == b, h_ref[...], 0.0),
                   axis=0, keepdims=True)
    hid = lax.dot_general(hsel, w1_ref[0], (((1,), (0,)), ((), ())),
                          preferred_element_type=jnp.float32) + b1_ref[0, 0]
    hid = jnp.maximum(hid, 0.0)
    contrib = lax.dot_general(hid, w2_ref[0], (((1,), (0,)), ((), ())),
                              preferred_element_type=jnp.float32)
    contrib = contrib + jnp.where(c == 0, 1.0, 0.0) * b2_ref[0]
    ciota = lax.broadcasted_iota(jnp.int32, (_B, _K), 1)
    v = jnp.sum(jnp.where((riota == b) & (ciota == k), vals_ref[...], 0.0))
    comb[...] = comb[...] + jnp.where(riota == b, 1.0, 0.0) * (v * contrib)

    @pl.when((t == _B * _K - 1) & (c == _NHC - 1))
    def _():
        xx = comb[...]
        mu = jnp.mean(xx, axis=-1, keepdims=True)
        var = jnp.mean((xx - mu) ** 2, axis=-1, keepdims=True)
        normed = (xx - mu) * lax.rsqrt(var + 1e-5) * lng_ref[...] + lnb_ref[...]
        out_ref[...] = lax.dot_general(
            normed, hw_ref[...], (((1,), (0,)), ((), ())),
            preferred_element_type=jnp.float32) + hb_ref[...]


def kernel(x, embed_w, gate_w, gate_b, w1, b1, w2, b2, ln_g, ln_b, head_w, head_b):
    x_resh = x.astype(jnp.int32).reshape(_NW, _NCH, _CH)

    emb_call = pl.kernel(
        _emb_body,
        out_type=jax.ShapeDtypeStruct((_NW, _D), jnp.float32),
        mesh=plsc.VectorSubcoreMesh(core_axis_name="c", subcore_axis_name="s"),
        scratch_types=[
            pltpu.VMEM((_NCH, _CH), jnp.int32),
            pltpu.VMEM((2, _CH, _D), jnp.float32),
            pltpu.VMEM((_D,), jnp.float32),
            pltpu.SemaphoreType.DMA,
            pltpu.SemaphoreType.DMA,
        ],
    )
    psums = emb_call(x_resh, embed_w)

    h, vals, ti = pl.pallas_call(
        _router_body,
        out_shape=[
            jax.ShapeDtypeStruct((_B, _D), jnp.float32),
            jax.ShapeDtypeStruct((_B, _K), jnp.float32),
            jax.ShapeDtypeStruct((_B, _K), jnp.int32),
        ],
    )(psums.reshape(_B, _NW // _B, _D), gate_w, gate_b.reshape(1, _E))

    grid_spec = pltpu.PrefetchScalarGridSpec(
        num_scalar_prefetch=1,
        grid=(_B * _K, _NHC),
        in_specs=[
            pl.BlockSpec((_B, _D), lambda t, c, ti: (0, 0)),
            pl.BlockSpec((_B, _K), lambda t, c, ti: (0, 0)),
            pl.BlockSpec((1, _D, _HCH), lambda t, c, ti: (ti[t], 0, c)),
            pl.BlockSpec((1, 1, 1, _HCH), lambda t, c, ti: (ti[t], c, 0, 0)),
            pl.BlockSpec((1, _HCH, _D), lambda t, c, ti: (ti[t], c, 0)),
            pl.BlockSpec((1, 1, _D), lambda t, c, ti: (ti[t], 0, 0)),
            pl.BlockSpec((1, _D), lambda t, c, ti: (0, 0)),
            pl.BlockSpec((1, _D), lambda t, c, ti: (0, 0)),
            pl.BlockSpec((_D, 20), lambda t, c, ti: (0, 0)),
            pl.BlockSpec((1, 20), lambda t, c, ti: (0, 0)),
        ],
        out_specs=pl.BlockSpec((_B, 20), lambda t, c, ti: (0, 0)),
        scratch_shapes=[pltpu.VMEM((_B, _D), jnp.float32)],
    )
    out_logits = pl.pallas_call(
        _moe_body,
        grid_spec=grid_spec,
        out_shape=jax.ShapeDtypeStruct((_B, 20), jnp.float32),
    )(ti.reshape(_B * _K), h, vals, w1, b1.reshape(_E, _NHC, 1, _HCH), w2,
      b2.reshape(_E, 1, _D),
      ln_g.reshape(1, _D), ln_b.reshape(1, _D), head_w, head_b.reshape(1, 20))

    return (out_logits, ti)


# speedup vs baseline: 4.2942x; 1.0097x over previous
import jax
import jax.numpy as jnp
from jax import lax
from jax.experimental import pallas as pl
from jax.experimental.pallas import tpu as pltpu
from jax.experimental.pallas import tpu_sc as plsc

_D = 768
_E = 64
_K = 2
_B = 4
_S = 8192
_NC = 2
_NS = 16
_NW = _NC * _NS
_TPW = (_B * _S) // _NW
_CH = 64
_NCH = _TPW // _CH
_LANES = 16
_DC = _D // _LANES
_HCH = 1536
_NHC = (2 * _D) // _HCH


def _emb_body(x_hbm, tab_hbm, out_hbm, idx_v, rows_v, acc_v, sem0, sem1):
    wid = lax.axis_index("s") * _NC + lax.axis_index("c")
    pltpu.sync_copy(x_hbm.at[wid], idx_v)

    def _zero(d, carry):
        acc_v[pl.ds(d * _LANES, _LANES)] = jnp.zeros((_LANES,), jnp.float32)
        return carry

    lax.fori_loop(0, _DC, _zero, 0)

    sems = (sem0, sem1)

    def _start(ch):
        return pltpu.async_copy(
            tab_hbm.at[idx_v.at[ch]], rows_v.at[ch % 2], sems[ch % 2])

    pending = {0: _start(0)}
    for ch in range(_NCH):
        if ch + 1 < _NCH:
            pending[ch + 1] = _start(ch + 1)
        pending.pop(ch).wait()
        buf = ch % 2

        def _accum(d, carry, buf=buf):
            base = d * _LANES
            lanes = [rows_v[buf, r, pl.ds(base, _LANES)] for r in range(_CH)]
            parts = []
            for g in range(4):
                a = lanes[g * (_CH // 4)]
                for r in range(g * (_CH // 4) + 1, (g + 1) * (_CH // 4)):
                    a = a + lanes[r]
                parts.append(a)
            acc_v[pl.ds(base, _LANES)] = (
                acc_v[pl.ds(base, _LANES)] + ((parts[0] + parts[1]) + (parts[2] + parts[3])))
            return carry

        lax.fori_loop(0, _DC, _accum, 0)

    pltpu.sync_copy(acc_v, out_hbm.at[wid // (_NW // _B), wid % (_NW // _B)])


def _router_body(ps_ref, gw_ref, gb_ref, h_ref, vals_ref, idx_ref):
    h = jnp.sum(ps_ref[...], axis=1) * (1.0 / _S)
    logits = lax.dot_general(h, gw_ref[...], (((1,), (0,)), ((), ())),
                             preferred_element_type=jnp.float32)
    logits = logits + gb_ref[...]
    m = jnp.max(logits, axis=-1, keepdims=True)
    ex = jnp.exp(logits - m)
    sm = ex / jnp.sum(ex, axis=-1, keepdims=True)
    iota = lax.broadcasted_iota(jnp.int32, (_B, _E), 1)
    v1 = jnp.max(sm, axis=-1, keepdims=True)
    i1 = jnp.min(jnp.where(sm == v1, iota, _E), axis=-1, keepdims=True)
    sm2 = jnp.where(iota == i1, -jnp.inf, sm)
    v2 = jnp.max(sm2, axis=-1, keepdims=True)
    i2 = jnp.min(jnp.where(sm2 == v2, iota, _E), axis=-1, keepdims=True)
    h_ref[...] = h
    vals_ref[...] = jnp.concatenate([v1, v2], axis=1)
    idx_ref[...] = jnp.concatenate([i1, i2], axis=1)


def _moe_body(ti_ref, h_ref, vals_ref, w1_ref, b1_ref, w2_ref, b2_ref,
              lng_ref, lnb_ref, hw_ref, hb_ref, out_ref, comb):
    t = pl.program_id(0)

    @pl.when(t == 0)
    def _():
        comb[...] = jnp.zeros((_B, _D), jnp.float32)

    b = t // _K
    k = t % _K
    riota = lax.broadcasted_iota(jnp.int32, (_B, 1), 0)
    hsel = jnp.sum(jnp.where(riota == b, h_ref[...], 0.0),
                   axis=0, keepdims=True)
    hid = lax.dot_general(hsel, w1_ref[0], (((1,), (0,)), ((), ())),
                          preferred_element_type=jnp.float32) + b1_ref[0]
    hid = jnp.maximum(hid, 0.0)
    contrib = lax.dot_general(hid, w2_ref[0], (((1,), (0,)), ((), ())),
                              preferred_element_type=jnp.float32)
    contrib = contrib + b2_ref[0]
    ciota = lax.broadcasted_iota(jnp.int32, (_B, _K), 1)
    v = jnp.sum(jnp.where((riota == b) & (ciota == k), vals_ref[...], 0.0))
    comb[...] = comb[...] + jnp.where(riota == b, 1.0, 0.0) * (v * contrib)

    @pl.when(t == _B * _K - 1)
    def _():
        xx = comb[...]
        mu = jnp.mean(xx, axis=-1, keepdims=True)
        var = jnp.mean((xx - mu) ** 2, axis=-1, keepdims=True)
        normed = (xx - mu) * lax.rsqrt(var + 1e-5) * lng_ref[...] + lnb_ref[...]
        out_ref[...] = lax.dot_general(
            normed, hw_ref[...], (((1,), (0,)), ((), ())),
            preferred_element_type=jnp.float32) + hb_ref[...]


def kernel(x, embed_w, gate_w, gate_b, w1, b1, w2, b2, ln_g, ln_b, head_w, head_b):
    x_resh = x.astype(jnp.int32).reshape(_NW, _NCH, _CH)

    emb_call = pl.kernel(
        _emb_body,
        out_type=jax.ShapeDtypeStruct((_B, _NW // _B, _D), jnp.float32),
        mesh=plsc.VectorSubcoreMesh(core_axis_name="c", subcore_axis_name="s"),
        scratch_types=[
            pltpu.VMEM((_NCH, _CH), jnp.int32),
            pltpu.VMEM((2, _CH, _D), jnp.float32),
            pltpu.VMEM((_D,), jnp.float32),
            pltpu.SemaphoreType.DMA,
            pltpu.SemaphoreType.DMA,
        ],
    )
    psums = emb_call(x_resh, embed_w)

    h, vals, ti = pl.pallas_call(
        _router_body,
        out_shape=[
            jax.ShapeDtypeStruct((_B, _D), jnp.float32),
            jax.ShapeDtypeStruct((_B, _K), jnp.float32),
            jax.ShapeDtypeStruct((_B, _K), jnp.int32),
        ],
    )(psums, gate_w, gate_b.reshape(1, _E))

    grid_spec = pltpu.PrefetchScalarGridSpec(
        num_scalar_prefetch=1,
        grid=(_B * _K,),
        in_specs=[
            pl.BlockSpec((_B, _D), lambda t, ti: (0, 0)),
            pl.BlockSpec((_B, _K), lambda t, ti: (0, 0)),
            pl.BlockSpec((1, _D, _HCH), lambda t, ti: (ti[t // _K, t % _K], 0, 0)),
            pl.BlockSpec((1, 1, _HCH), lambda t, ti: (ti[t // _K, t % _K], 0, 0)),
            pl.BlockSpec((1, _HCH, _D), lambda t, ti: (ti[t // _K, t % _K], 0, 0)),
            pl.BlockSpec((1, 1, _D), lambda t, ti: (ti[t // _K, t % _K], 0, 0)),
            pl.BlockSpec((1, _D), lambda t, ti: (0, 0)),
            pl.BlockSpec((1, _D), lambda t, ti: (0, 0)),
            pl.BlockSpec((_D, 20), lambda t, ti: (0, 0)),
            pl.BlockSpec((1, 20), lambda t, ti: (0, 0)),
        ],
        out_specs=pl.BlockSpec((_B, 20), lambda t, ti: (0, 0)),
        scratch_shapes=[pltpu.VMEM((_B, _D), jnp.float32)],
    )
    out_logits = pl.pallas_call(
        _moe_body,
        grid_spec=grid_spec,
        out_shape=jax.ShapeDtypeStruct((_B, 20), jnp.float32),
    )(ti, h, vals, w1, b1.reshape(_E, 1, _HCH), w2, b2.reshape(_E, 1, _D),
      ln_g.reshape(1, _D), ln_b.reshape(1, _D), head_w, head_b.reshape(1, 20))

    return (out_logits, ti)


# speedup vs baseline: 4.3017x; 1.0017x over previous
import jax
import jax.numpy as jnp
from jax import lax
from jax.experimental import pallas as pl
from jax.experimental.pallas import tpu as pltpu
from jax.experimental.pallas import tpu_sc as plsc

_D = 768
_E = 64
_K = 2
_B = 4
_S = 8192
_NC = 2
_NS = 16
_NW = _NC * _NS
_TPW = (_B * _S) // _NW
_CH = 64
_NCH = _TPW // _CH
_LANES = 16
_DC = _D // _LANES
_H = 2 * _D


def _emb_body(x_hbm, tab_hbm, out_hbm, idx_v, rows_v, acc_v, sem0, sem1):
    wid = lax.axis_index("s") * _NC + lax.axis_index("c")
    nwb = _NW // _B
    b = wid // nwb
    pltpu.sync_copy(x_hbm.at[b, pl.ds((wid % nwb) * _TPW, _TPW)], idx_v)

    sems = (sem0, sem1)

    def _start(ch):
        return pltpu.async_copy(
            tab_hbm.at[idx_v.at[pl.ds(ch * _CH, _CH)]],
            rows_v.at[ch % 2], sems[ch % 2])

    pending = {0: _start(0)}

    def _zero(d, carry):
        acc_v[pl.ds(d * _LANES, _LANES)] = jnp.zeros((_LANES,), jnp.float32)
        return carry

    lax.fori_loop(0, _DC, _zero, 0)

    for ch in range(_NCH):
        if ch + 1 < _NCH:
            pending[ch + 1] = _start(ch + 1)
        pending.pop(ch).wait()
        buf = ch % 2

        def _accum(d, carry, buf=buf):
            base = d * _LANES
            lanes = [rows_v[buf, r, pl.ds(base, _LANES)] for r in range(_CH)]
            parts = []
            for g in range(4):
                a = lanes[g * (_CH // 4)]
                for r in range(g * (_CH // 4) + 1, (g + 1) * (_CH // 4)):
                    a = a + lanes[r]
                parts.append(a)
            acc_v[pl.ds(base, _LANES)] = (
                acc_v[pl.ds(base, _LANES)] + ((parts[0] + parts[1]) + (parts[2] + parts[3])))
            return carry

        lax.fori_loop(0, _DC, _accum, 0)

    pltpu.sync_copy(acc_v, out_hbm.at[wid // nwb, wid % nwb])


def _router_body(ps_ref, gw_ref, gb_ref, h_ref, vals_ref, idx_ref):
    h = jnp.sum(ps_ref[...], axis=1) * (1.0 / _S)
    logits = lax.dot_general(h, gw_ref[...], (((1,), (0,)), ((), ())),
                             preferred_element_type=jnp.float32)
    logits = logits + gb_ref[...]
    m = jnp.max(logits, axis=-1, keepdims=True)
    ex = jnp.exp(logits - m)
    sm = ex / jnp.sum(ex, axis=-1, keepdims=True)
    iota = lax.broadcasted_iota(jnp.int32, (_B, _E), 1)
    v1 = jnp.max(sm, axis=-1, keepdims=True)
    i1 = jnp.min(jnp.where(sm == v1, iota, _E), axis=-1, keepdims=True)
    sm2 = jnp.where(iota == i1, -jnp.inf, sm)
    v2 = jnp.max(sm2, axis=-1, keepdims=True)
    i2 = jnp.min(jnp.where(sm2 == v2, iota, _E), axis=-1, keepdims=True)
    h_ref[...] = h
    vals_ref[...] = jnp.concatenate([v1, v2], axis=1)
    idx_ref[...] = jnp.concatenate([i1, i2], axis=1)


def _moe_body(ti_ref, vals_ref, h_ref, b1_ref, b2_ref, lng_ref, lnb_ref,
              hw_ref, hb_ref, w1_any, w2_any, out_ref,
              w1buf, w2buf, sem1, sem2):
    npairs = _B * _K
    eiota1 = lax.broadcasted_iota(jnp.int32, (_E, 1), 0)

    def _start(p):
        e = ti_ref[p // _K, p % _K]
        s = p % 2
        c1 = pltpu.make_async_copy(w1_any.at[e], w1buf.at[s], sem1.at[s])
        c2 = pltpu.make_async_copy(w2_any.at[e], w2buf.at[s], sem2.at[s])
        c1.start()
        c2.start()
        return (c1, c2)

    pending = {0: _start(0)}
    outs = []
    for p in range(npairs):
        if p + 1 < npairs:
            pending[p + 1] = _start(p + 1)
        c1, c2 = pending.pop(p)
        c1.wait()
        c2.wait()
        s = p % 2
        e = ti_ref[p // _K, p % _K]
        b1row = jnp.sum(jnp.where(eiota1 == e, b1_ref[...], 0.0),
                        axis=0, keepdims=True)
        b2row = jnp.sum(jnp.where(eiota1 == e, b2_ref[...], 0.0),
                        axis=0, keepdims=True)
        hsel = h_ref[p // _K:p // _K + 1, :]
        hid = lax.dot_general(hsel, w1buf[s], (((1,), (0,)), ((), ())),
                              preferred_element_type=jnp.float32) + b1row
        hid = jnp.maximum(hid, 0.0)
        contrib = lax.dot_general(hid, w2buf[s], (((1,), (0,)), ((), ())),
                                  preferred_element_type=jnp.float32) + b2row
        outs.append(vals_ref[p // _K, p % _K] * contrib)

    comb = jnp.concatenate(
        [outs[2 * b] + outs[2 * b + 1] for b in range(_B)], axis=0)
    mu = jnp.mean(comb, axis=-1, keepdims=True)
    var = jnp.mean((comb - mu) ** 2, axis=-1, keepdims=True)
    normed = (comb - mu) * lax.rsqrt(var + 1e-5) * lng_ref[...] + lnb_ref[...]
    out_ref[...] = lax.dot_general(
        normed, hw_ref[...], (((1,), (0,)), ((), ())),
        preferred_element_type=jnp.float32) + hb_ref[...]


def kernel(x, embed_w, gate_w, gate_b, w1, b1, w2, b2, ln_g, ln_b, head_w, head_b):
    emb_call = pl.kernel(
        _emb_body,
        out_type=jax.ShapeDtypeStruct((_B, _NW // _B, _D), jnp.float32),
        mesh=plsc.VectorSubcoreMesh(core_axis_name="c", subcore_axis_name="s"),
        scratch_types=[
            pltpu.VMEM((_TPW,), jnp.int32),
            pltpu.VMEM((2, _CH, _D), jnp.float32),
            pltpu.VMEM((_D,), jnp.float32),
            pltpu.SemaphoreType.DMA,
            pltpu.SemaphoreType.DMA,
        ],
    )
    psums = emb_call(x.astype(jnp.int32), embed_w)

    h, vals, ti = pl.pallas_call(
        _router_body,
        out_shape=[
            jax.ShapeDtypeStruct((_B, _D), jnp.float32),
            jax.ShapeDtypeStruct((_B, _K), jnp.float32),
            jax.ShapeDtypeStruct((_B, _K), jnp.int32),
        ],
    )(psums, gate_w, gate_b.reshape(1, _E))

    out_logits = pl.pallas_call(
        _moe_body,
        in_specs=[
            pl.BlockSpec(memory_space=pltpu.SMEM),
            pl.BlockSpec(memory_space=pltpu.SMEM),
            pl.BlockSpec(memory_space=pltpu.VMEM),
            pl.BlockSpec(memory_space=pltpu.VMEM),
            pl.BlockSpec(memory_space=pltpu.VMEM),
            pl.BlockSpec(memory_space=pltpu.VMEM),
            pl.BlockSpec(memory_space=pltpu.VMEM),
            pl.BlockSpec(memory_space=pltpu.VMEM),
            pl.BlockSpec(memory_space=pltpu.VMEM),
            pl.BlockSpec(memory_space=pl.ANY),
            pl.BlockSpec(memory_space=pl.ANY),
        ],
        out_shape=jax.ShapeDtypeStruct((_B, 20), jnp.float32),
        scratch_shapes=[
            pltpu.VMEM((2, _D, _H), jnp.float32),
            pltpu.VMEM((2, _H, _D), jnp.float32),
            pltpu.SemaphoreType.DMA((2,)),
            pltpu.SemaphoreType.DMA((2,)),
        ],
    )(ti, vals, h, b1, b2,
      ln_g.reshape(1, _D), ln_b.reshape(1, _D), head_w, head_b.reshape(1, 20),
      w1, w2)

    return (out_logits, ti)


# speedup vs baseline: 4.3049x; 1.0007x over previous
import jax
import jax.numpy as jnp
from jax import lax
from jax.experimental import pallas as pl
from jax.experimental.pallas import tpu as pltpu
from jax.experimental.pallas import tpu_sc as plsc

_D = 768
_E = 64
_K = 2
_B = 4
_S = 8192
_NC = 2
_NS = 16
_NW = _NC * _NS
_TPW = (_B * _S) // _NW
_CH = 64
_NCH = _TPW // _CH
_LANES = 16
_DC = _D // _LANES
_H = 2 * _D


def _emb_body(x_hbm, tab_hbm, out_hbm, idx_v, rows_v, acc_v, sem0, sem1):
    wid = lax.axis_index("s") * _NC + lax.axis_index("c")
    nwb = _NW // _B
    b = wid // nwb
    pltpu.sync_copy(x_hbm.at[b, pl.ds((wid % nwb) * _TPW, _TPW)], idx_v)

    sems = (sem0, sem1)

    def _start(ch):
        return pltpu.async_copy(
            tab_hbm.at[idx_v.at[pl.ds(ch * _CH, _CH)]],
            rows_v.at[ch % 2], sems[ch % 2])

    pending = {0: _start(0)}

    def _zero(d, carry):
        acc_v[pl.ds(d * _LANES, _LANES)] = jnp.zeros((_LANES,), jnp.float32)
        return carry

    lax.fori_loop(0, _DC, _zero, 0)

    for ch in range(_NCH):
        if ch + 1 < _NCH:
            pending[ch + 1] = _start(ch + 1)
        pending.pop(ch).wait()
        buf = ch % 2

        def _accum(d, carry, buf=buf):
            base = d * _LANES
            lanes = [rows_v[buf, r, pl.ds(base, _LANES)] for r in range(_CH)]
            parts = []
            for g in range(4):
                a = lanes[g * (_CH // 4)]
                for r in range(g * (_CH // 4) + 1, (g + 1) * (_CH // 4)):
                    a = a + lanes[r]
                parts.append(a)
            acc_v[pl.ds(base, _LANES)] = (
                acc_v[pl.ds(base, _LANES)] + ((parts[0] + parts[1]) + (parts[2] + parts[3])))
            return carry

        lax.fori_loop(0, _DC, _accum, 0)

    pltpu.sync_copy(acc_v, out_hbm.at[wid // nwb, wid % nwb])


def _router_body(ps_ref, gw_ref, gb_ref, h_ref, vals_ref, idx_ref):
    h = jnp.sum(ps_ref[...], axis=1) * (1.0 / _S)
    logits = lax.dot_general(h, gw_ref[...], (((1,), (0,)), ((), ())),
                             preferred_element_type=jnp.float32)
    logits = logits + gb_ref[...]
    m = jnp.max(logits, axis=-1, keepdims=True)
    ex = jnp.exp(logits - m)
    sm = ex / jnp.sum(ex, axis=-1, keepdims=True)
    iota = lax.broadcasted_iota(jnp.int32, (_B, _E), 1)
    v1 = jnp.max(sm, axis=-1, keepdims=True)
    i1 = jnp.min(jnp.where(sm == v1, iota, _E), axis=-1, keepdims=True)
    sm2 = jnp.where(iota == i1, -jnp.inf, sm)
    v2 = jnp.max(sm2, axis=-1, keepdims=True)
    i2 = jnp.min(jnp.where(sm2 == v2, iota, _E), axis=-1, keepdims=True)
    h_ref[...] = h
    vals_ref[...] = jnp.concatenate([v1, v2], axis=1)
    idx_ref[...] = jnp.concatenate([i1, i2], axis=1)


def _moe_body(ti_ref, vals_ref, h_ref, b1_ref, b2_ref, lng_ref, lnb_ref,
              hw_ref, hb_ref, w1_any, w2_any, out_ref,
              w1buf, w2buf, sem1, sem2):
    npairs = _B * _K
    eiota1 = lax.broadcasted_iota(jnp.int32, (_E, 1), 0)

    def _start(p):
        e = ti_ref[p // _K, p % _K]
        s = p % 4
        c1 = pltpu.make_async_copy(w1_any.at[e], w1buf.at[s], sem1.at[s])
        c2 = pltpu.make_async_copy(w2_any.at[e], w2buf.at[s], sem2.at[s])
        c1.start()
        c2.start()
        return (c1, c2)

    pending = {p: _start(p) for p in range(3)}
    outs = []
    for p in range(npairs):
        if p + 3 < npairs:
            pending[p + 3] = _start(p + 3)
        c1, c2 = pending.pop(p)
        c1.wait()
        c2.wait()
        s = p % 4
        e = ti_ref[p // _K, p % _K]
        b1row = jnp.sum(jnp.where(eiota1 == e, b1_ref[...], 0.0),
                        axis=0, keepdims=True)
        b2row = jnp.sum(jnp.where(eiota1 == e, b2_ref[...], 0.0),
                        axis=0, keepdims=True)
        hsel = h_ref[p // _K:p // _K + 1, :]
        hid = lax.dot_general(hsel, w1buf[s], (((1,), (0,)), ((), ())),
                              preferred_element_type=jnp.float32) + b1row
        hid = jnp.maximum(hid, 0.0)
        contrib = lax.dot_general(hid, w2buf[s], (((1,), (0,)), ((), ())),
                                  preferred_element_type=jnp.float32) + b2row
        outs.append(vals_ref[p // _K, p % _K] * contrib)

    comb = jnp.concatenate(
        [outs[2 * b] + outs[2 * b + 1] for b in range(_B)], axis=0)
    mu = jnp.mean(comb, axis=-1, keepdims=True)
    var = jnp.mean((comb - mu) ** 2, axis=-1, keepdims=True)
    normed = (comb - mu) * lax.rsqrt(var + 1e-5) * lng_ref[...] + lnb_ref[...]
    out_ref[...] = lax.dot_general(
        normed, hw_ref[...], (((1,), (0,)), ((), ())),
        preferred_element_type=jnp.float32) + hb_ref[...]


def kernel(x, embed_w, gate_w, gate_b, w1, b1, w2, b2, ln_g, ln_b, head_w, head_b):
    emb_call = pl.kernel(
        _emb_body,
        out_type=jax.ShapeDtypeStruct((_B, _NW // _B, _D), jnp.float32),
        mesh=plsc.VectorSubcoreMesh(core_axis_name="c", subcore_axis_name="s"),
        scratch_types=[
            pltpu.VMEM((_TPW,), jnp.int32),
            pltpu.VMEM((2, _CH, _D), jnp.float32),
            pltpu.VMEM((_D,), jnp.float32),
            pltpu.SemaphoreType.DMA,
            pltpu.SemaphoreType.DMA,
        ],
    )
    psums = emb_call(x.astype(jnp.int32), embed_w)

    h, vals, ti = pl.pallas_call(
        _router_body,
        out_shape=[
            jax.ShapeDtypeStruct((_B, _D), jnp.float32),
            jax.ShapeDtypeStruct((_B, _K), jnp.float32),
            jax.ShapeDtypeStruct((_B, _K), jnp.int32),
        ],
    )(psums, gate_w, gate_b.reshape(1, _E))

    out_logits = pl.pallas_call(
        _moe_body,
        in_specs=[
            pl.BlockSpec(memory_space=pltpu.SMEM),
            pl.BlockSpec(memory_space=pltpu.SMEM),
            pl.BlockSpec(memory_space=pltpu.VMEM),
            pl.BlockSpec(memory_space=pltpu.VMEM),
            pl.BlockSpec(memory_space=pltpu.VMEM),
            pl.BlockSpec(memory_space=pltpu.VMEM),
            pl.BlockSpec(memory_space=pltpu.VMEM),
            pl.BlockSpec(memory_space=pltpu.VMEM),
            pl.BlockSpec(memory_space=pltpu.VMEM),
            pl.BlockSpec(memory_space=pl.ANY),
            pl.BlockSpec(memory_space=pl.ANY),
        ],
        out_shape=jax.ShapeDtypeStruct((_B, 20), jnp.float32),
        scratch_shapes=[
            pltpu.VMEM((4, _D, _H), jnp.float32),
            pltpu.VMEM((4, _H, _D), jnp.float32),
            pltpu.SemaphoreType.DMA((4,)),
            pltpu.SemaphoreType.DMA((4,)),
        ],
    )(ti, vals, h, b1, b2,
      ln_g.reshape(1, _D), ln_b.reshape(1, _D), head_w, head_b.reshape(1, 20),
      w1, w2)

    return (out_logits, ti)


# speedup vs baseline: 4.3085x; 1.0009x over previous
import jax
import jax.numpy as jnp
from jax import lax
from jax.experimental import pallas as pl
from jax.experimental.pallas import tpu as pltpu
from jax.experimental.pallas import tpu_sc as plsc

_D = 768
_E = 64
_K = 2
_B = 4
_S = 8192
_NC = 2
_NS = 16
_NW = _NC * _NS
_TPW = (_B * _S) // _NW
_CH = 64
_NCH = _TPW // _CH
_LANES = 16
_DC = _D // _LANES
_H = 2 * _D


def _emb_body(x_hbm, tab_hbm, out_hbm, idx_v, rows_v, acc_v, sem0, sem1):
    wid = lax.axis_index("s") * _NC + lax.axis_index("c")
    nwb = _NW // _B
    b = wid // nwb
    pltpu.sync_copy(x_hbm.at[b, pl.ds((wid % nwb) * _TPW, _TPW)], idx_v)

    sems = (sem0, sem1)

    def _start(ch):
        return pltpu.async_copy(
            tab_hbm.at[idx_v.at[pl.ds(ch * _CH, _CH)]],
            rows_v.at[ch % 2], sems[ch % 2])

    pending = {0: _start(0)}

    def _zero(d, carry):
        acc_v[pl.ds(d * _LANES, _LANES)] = jnp.zeros((_LANES,), jnp.float32)
        return carry

    lax.fori_loop(0, _DC, _zero, 0)

    for ch in range(_NCH):
        if ch + 1 < _NCH:
            pending[ch + 1] = _start(ch + 1)
        pending.pop(ch).wait()
        buf = ch % 2

        def _accum(d, carry, buf=buf):
            base = d * _LANES
            lanes = [rows_v[buf, r, pl.ds(base, _LANES)] for r in range(_CH)]
            parts = []
            for g in range(4):
                a = lanes[g * (_CH // 4)]
                for r in range(g * (_CH // 4) + 1, (g + 1) * (_CH // 4)):
                    a = a + lanes[r]
                parts.append(a)
            acc_v[pl.ds(base, _LANES)] = (
                acc_v[pl.ds(base, _LANES)] + ((parts[0] + parts[1]) + (parts[2] + parts[3])))
            return carry

        lax.fori_loop(0, _DC, _accum, 0)

    pltpu.sync_copy(acc_v, out_hbm.at[wid // nwb, wid % nwb])


def _router_body(ps_ref, gw_ref, gb_ref, h_ref, vals_ref, idx_ref):
    h = jnp.sum(ps_ref[...], axis=1) * (1.0 / _S)
    logits = lax.dot_general(h, gw_ref[...], (((1,), (0,)), ((), ())),
                             preferred_element_type=jnp.float32)
    logits = logits + gb_ref[...]
    m = jnp.max(logits, axis=-1, keepdims=True)
    ex = jnp.exp(logits - m)
    sm = ex / jnp.sum(ex, axis=-1, keepdims=True)
    iota = lax.broadcasted_iota(jnp.int32, (_B, _E), 1)
    v1 = jnp.max(sm, axis=-1, keepdims=True)
    i1 = jnp.min(jnp.where(sm == v1, iota, _E), axis=-1, keepdims=True)
    sm2 = jnp.where(iota == i1, -jnp.inf, sm)
    v2 = jnp.max(sm2, axis=-1, keepdims=True)
    i2 = jnp.min(jnp.where(sm2 == v2, iota, _E), axis=-1, keepdims=True)
    h_ref[...] = h
    vals_ref[...] = jnp.concatenate([v1, v2], axis=1)
    idx_ref[...] = jnp.concatenate([i1, i2], axis=1)


def _moe_body(ti_ref, vals_ref, h_ref, b1_ref, b2_ref, lng_ref, lnb_ref,
              hw_ref, hb_ref, w1_any, w2_any, out_ref,
              w1buf, w2buf, sem1, sem2):
    npairs = _B * _K
    eiota1 = lax.broadcasted_iota(jnp.int32, (_E, 1), 0)

    def _start(p):
        e = ti_ref[p // _K, p % _K]
        s = p % 4
        cs = []
        for lo, n in ((0, _D // 2), (_D // 2, _D // 2)):
            cs.append(pltpu.make_async_copy(
                w1_any.at[e, pl.ds(lo, n)], w1buf.at[s, pl.ds(lo, n)],
                sem1.at[s]))
        for lo, n in ((0, _H // 2), (_H // 2, _H // 2)):
            cs.append(pltpu.make_async_copy(
                w2_any.at[e, pl.ds(lo, n)], w2buf.at[s, pl.ds(lo, n)],
                sem2.at[s]))
        for c in cs:
            c.start()
        return cs

    pending = {p: _start(p) for p in range(3)}
    outs = []
    for p in range(npairs):
        if p + 3 < npairs:
            pending[p + 3] = _start(p + 3)
        for c in pending.pop(p):
            c.wait()
        s = p % 4
        e = ti_ref[p // _K, p % _K]
        b1row = jnp.sum(jnp.where(eiota1 == e, b1_ref[...], 0.0),
                        axis=0, keepdims=True)
        b2row = jnp.sum(jnp.where(eiota1 == e, b2_ref[...], 0.0),
                        axis=0, keepdims=True)
        hsel = h_ref[p // _K:p // _K + 1, :]
        hid = lax.dot_general(hsel, w1buf[s], (((1,), (0,)), ((), ())),
                              preferred_element_type=jnp.float32) + b1row
        hid = jnp.maximum(hid, 0.0)
        contrib = lax.dot_general(hid, w2buf[s], (((1,), (0,)), ((), ())),
                                  preferred_element_type=jnp.float32) + b2row
        outs.append(vals_ref[p // _K, p % _K] * contrib)

    comb = jnp.concatenate(
        [outs[2 * b] + outs[2 * b + 1] for b in range(_B)], axis=0)
    mu = jnp.mean(comb, axis=-1, keepdims=True)
    var = jnp.mean((comb - mu) ** 2, axis=-1, keepdims=True)
    normed = (comb - mu) * lax.rsqrt(var + 1e-5) * lng_ref[...] + lnb_ref[...]
    out_ref[...] = lax.dot_general(
        normed, hw_ref[...], (((1,), (0,)), ((), ())),
        preferred_element_type=jnp.float32) + hb_ref[...]


def kernel(x, embed_w, gate_w, gate_b, w1, b1, w2, b2, ln_g, ln_b, head_w, head_b):
    emb_call = pl.kernel(
        _emb_body,
        out_type=jax.ShapeDtypeStruct((_B, _NW // _B, _D), jnp.float32),
        mesh=plsc.VectorSubcoreMesh(core_axis_name="c", subcore_axis_name="s"),
        scratch_types=[
            pltpu.VMEM((_TPW,), jnp.int32),
            pltpu.VMEM((2, _CH, _D), jnp.float32),
            pltpu.VMEM((_D,), jnp.float32),
            pltpu.SemaphoreType.DMA,
            pltpu.SemaphoreType.DMA,
        ],
    )
    psums = emb_call(x.astype(jnp.int32), embed_w)

    h, vals, ti = pl.pallas_call(
        _router_body,
        out_shape=[
            jax.ShapeDtypeStruct((_B, _D), jnp.float32),
            jax.ShapeDtypeStruct((_B, _K), jnp.float32),
            jax.ShapeDtypeStruct((_B, _K), jnp.int32),
        ],
    )(psums, gate_w, gate_b.reshape(1, _E))

    out_logits = pl.pallas_call(
        _moe_body,
        in_specs=[
            pl.BlockSpec(memory_space=pltpu.SMEM),
            pl.BlockSpec(memory_space=pltpu.SMEM),
            pl.BlockSpec(memory_space=pltpu.VMEM),
            pl.BlockSpec(memory_space=pltpu.VMEM),
            pl.BlockSpec(memory_space=pltpu.VMEM),
            pl.BlockSpec(memory_space=pltpu.VMEM),
            pl.BlockSpec(memory_space=pltpu.VMEM),
            pl.BlockSpec(memory_space=pltpu.VMEM),
            pl.BlockSpec(memory_space=pltpu.VMEM),
            pl.BlockSpec(memory_space=pl.ANY),
            pl.BlockSpec(memory_space=pl.ANY),
        ],
        out_shape=jax.ShapeDtypeStruct((_B, 20), jnp.float32),
        scratch_shapes=[
            pltpu.VMEM((4, _D, _H), jnp.float32),
            pltpu.VMEM((4, _H, _D), jnp.float32),
            pltpu.SemaphoreType.DMA((4,)),
            pltpu.SemaphoreType.DMA((4,)),
        ],
    )(ti, vals, h, b1, b2,
      ln_g.reshape(1, _D), ln_b.reshape(1, _D), head_w, head_b.reshape(1, 20),
      w1, w2)

    return (out_logits, ti)


# speedup vs baseline: 4.9137x; 1.1405x over previous
import jax
import jax.numpy as jnp
from jax import lax
from jax.experimental import pallas as pl
from jax.experimental.pallas import tpu as pltpu
from jax.experimental.pallas import tpu_sc as plsc

_D = 768
_E = 64
_K = 2
_B = 4
_S = 8192
_NC = 2
_NS = 16
_NW = _NC * _NS
_TPW = (_B * _S) // _NW
_CH = 64
_NCH = _TPW // _CH
_LANES = 16
_DC = _D // _LANES
_H = 2 * _D


def _emb_body(x_hbm, tab_hbm, out_hbm, idx_v, rows_v, acc_v, sem):
    wid = lax.axis_index("s") * _NC + lax.axis_index("c")
    nwb = _NW // _B
    b = wid // nwb
    pltpu.sync_copy(x_hbm.at[b, pl.ds((wid % nwb) * _TPW, _TPW)], idx_v)

    def _gather(ch, buf):
        return pltpu.make_async_copy(
            tab_hbm.at[idx_v.at[pl.ds(ch * _CH, _CH)]],
            rows_v.at[buf], sem.at[buf])

    _gather(0, 0).start()
    _gather(1, 1).start()

    def _zero(d, carry):
        acc_v[pl.ds(d * _LANES, _LANES)] = jnp.zeros((_LANES,), jnp.float32)
        return carry

    lax.fori_loop(0, _DC, _zero, 0)

    def _chunk(ch, carry):
        buf = ch % 2
        _gather(ch, buf).wait()

        @pl.when(ch + 2 < _NCH)
        def _():
            _gather(ch + 2, buf).start()

        def _accum(d, carry2):
            base = d * _LANES
            lanes = [rows_v[buf, r, pl.ds(base, _LANES)] for r in range(_CH)]
            parts = []
            for g in range(4):
                a = lanes[g * (_CH // 4)]
                for r in range(g * (_CH // 4) + 1, (g + 1) * (_CH // 4)):
                    a = a + lanes[r]
                parts.append(a)
            acc_v[pl.ds(base, _LANES)] = (
                acc_v[pl.ds(base, _LANES)] + ((parts[0] + parts[1]) + (parts[2] + parts[3])))
            return carry2

        lax.fori_loop(0, _DC, _accum, 0)
        return carry

    lax.fori_loop(0, _NCH, _chunk, 0)

    pltpu.sync_copy(acc_v, out_hbm.at[wid // nwb, wid % nwb])


def _router_body(ps_ref, gw_ref, gb_ref, h_ref, vals_ref, idx_ref):
    h = jnp.sum(ps_ref[...], axis=1) * (1.0 / _S)
    logits = lax.dot_general(h, gw_ref[...], (((1,), (0,)), ((), ())),
                             preferred_element_type=jnp.float32)
    logits = logits + gb_ref[...]
    m = jnp.max(logits, axis=-1, keepdims=True)
    ex = jnp.exp(logits - m)
    sm = ex / jnp.sum(ex, axis=-1, keepdims=True)
    iota = lax.broadcasted_iota(jnp.int32, (_B, _E), 1)
    v1 = jnp.max(sm, axis=-1, keepdims=True)
    i1 = jnp.min(jnp.where(sm == v1, iota, _E), axis=-1, keepdims=True)
    sm2 = jnp.where(iota == i1, -jnp.inf, sm)
    v2 = jnp.max(sm2, axis=-1, keepdims=True)
    i2 = jnp.min(jnp.where(sm2 == v2, iota, _E), axis=-1, keepdims=True)
    h_ref[...] = h
    vals_ref[...] = jnp.concatenate([v1, v2], axis=1)
    idx_ref[...] = jnp.concatenate([i1, i2], axis=1)


def _moe_body(ti_ref, vals_ref, h_ref, b1_ref, b2_ref, lng_ref, lnb_ref,
              hw_ref, hb_ref, w1_any, w2_any, out_ref,
              w1buf, w2buf, sem1, sem2):
    npairs = _B * _K
    eiota1 = lax.broadcasted_iota(jnp.int32, (_E, 1), 0)

    def _start(p):
        e = ti_ref[p // _K, p % _K]
        s = p % 4
        cs = []
        for lo, n in ((0, _D // 2), (_D // 2, _D // 2)):
            cs.append(pltpu.make_async_copy(
                w1_any.at[e, pl.ds(lo, n)], w1buf.at[s, pl.ds(lo, n)],
                sem1.at[s]))
        for lo, n in ((0, _H // 2), (_H // 2, _H // 2)):
            cs.append(pltpu.make_async_copy(
                w2_any.at[e, pl.ds(lo, n)], w2buf.at[s, pl.ds(lo, n)],
                sem2.at[s]))
        for c in cs:
            c.start()
        return cs

    pending = {p: _start(p) for p in range(3)}
    outs = []
    for p in range(npairs):
        if p + 3 < npairs:
            pending[p + 3] = _start(p + 3)
        for c in pending.pop(p):
            c.wait()
        s = p % 4
        e = ti_ref[p // _K, p % _K]
        b1row = jnp.sum(jnp.where(eiota1 == e, b1_ref[...], 0.0),
                        axis=0, keepdims=True)
        b2row = jnp.sum(jnp.where(eiota1 == e, b2_ref[...], 0.0),
                        axis=0, keepdims=True)
        hsel = h_ref[p // _K:p // _K + 1, :]
        hid = lax.dot_general(hsel, w1buf[s], (((1,), (0,)), ((), ())),
                              preferred_element_type=jnp.float32) + b1row
        hid = jnp.maximum(hid, 0.0)
        contrib = lax.dot_general(hid, w2buf[s], (((1,), (0,)), ((), ())),
                                  preferred_element_type=jnp.float32) + b2row
        outs.append(vals_ref[p // _K, p % _K] * contrib)

    comb = jnp.concatenate(
        [outs[2 * b] + outs[2 * b + 1] for b in range(_B)], axis=0)
    mu = jnp.mean(comb, axis=-1, keepdims=True)
    var = jnp.mean((comb - mu) ** 2, axis=-1, keepdims=True)
    normed = (comb - mu) * lax.rsqrt(var + 1e-5) * lng_ref[...] + lnb_ref[...]
    out_ref[...] = lax.dot_general(
        normed, hw_ref[...], (((1,), (0,)), ((), ())),
        preferred_element_type=jnp.float32) + hb_ref[...]


def kernel(x, embed_w, gate_w, gate_b, w1, b1, w2, b2, ln_g, ln_b, head_w, head_b):
    emb_call = pl.kernel(
        _emb_body,
        out_type=jax.ShapeDtypeStruct((_B, _NW // _B, _D), jnp.float32),
        mesh=plsc.VectorSubcoreMesh(core_axis_name="c", subcore_axis_name="s"),
        scratch_types=[
            pltpu.VMEM((_TPW,), jnp.int32),
            pltpu.VMEM((2, _CH, _D), jnp.float32),
            pltpu.VMEM((_D,), jnp.float32),
            pltpu.SemaphoreType.DMA((2,)),
        ],
    )
    psums = emb_call(x.astype(jnp.int32), embed_w)

    h, vals, ti = pl.pallas_call(
        _router_body,
        out_shape=[
            jax.ShapeDtypeStruct((_B, _D), jnp.float32),
            jax.ShapeDtypeStruct((_B, _K), jnp.float32),
            jax.ShapeDtypeStruct((_B, _K), jnp.int32),
        ],
    )(psums, gate_w, gate_b.reshape(1, _E))

    out_logits = pl.pallas_call(
        _moe_body,
        in_specs=[
            pl.BlockSpec(memory_space=pltpu.SMEM),
            pl.BlockSpec(memory_space=pltpu.SMEM),
            pl.BlockSpec(memory_space=pltpu.VMEM),
            pl.BlockSpec(memory_space=pltpu.VMEM),
            pl.BlockSpec(memory_space=pltpu.VMEM),
            pl.BlockSpec(memory_space=pltpu.VMEM),
            pl.BlockSpec(memory_space=pltpu.VMEM),
            pl.BlockSpec(memory_space=pltpu.VMEM),
            pl.BlockSpec(memory_space=pltpu.VMEM),
            pl.BlockSpec(memory_space=pl.ANY),
            pl.BlockSpec(memory_space=pl.ANY),
        ],
        out_shape=jax.ShapeDtypeStruct((_B, 20), jnp.float32),
        scratch_shapes=[
            pltpu.VMEM((4, _D, _H), jnp.float32),
            pltpu.VMEM((4, _H, _D), jnp.float32),
            pltpu.SemaphoreType.DMA((4,)),
            pltpu.SemaphoreType.DMA((4,)),
        ],
    )(ti, vals, h, b1, b2,
      ln_g.reshape(1, _D), ln_b.reshape(1, _D), head_w, head_b.reshape(1, 20),
      w1, w2)

    return (out_logits, ti)
